# baseline (speedup 1.0000x reference)
import functools

import numpy as np
import jax
import jax.numpy as jnp
from jax import lax
from jax.experimental import pallas as pl
from jax.experimental.pallas import tpu as pltpu
from jax.experimental.pallas import tpu_sc as plsc

_NUM_LEVELS = 16
_FEAT = 2
_T = 1 << 19
_BASE_RES = 16
_SCALE = float(np.exp(np.log(4096.0 / 16.0) / (_NUM_LEVELS - 1)))
_RES = [int(np.floor(_BASE_RES * _SCALE ** l)) for l in range(_NUM_LEVELS)]
_P1 = np.uint32(2654435761)
_P2 = np.uint32(805459861)

_NC = 2
_NS = 16
_NW = _NC * _NS
_LANES = 16

_P = 512
_GROUPS = _P // _LANES
_NROWS = _P * 8
_IDX_ROWS = _NROWS // 128


def _encode_body(texc_hbm, tbl_hbm, out_hbm, txyz, xb, yb, zb, idx2, w2,
                 rows2, penc, sem0, sem1):
    n_pts = texc_hbm.shape[0] // 3
    ppw = n_pts // _NW
    chunks = ppw // _P

    cid = lax.axis_index("c")
    sid = lax.axis_index("s")
    wid = sid * _NC + cid

    iota = lax.iota(jnp.int32, _LANES)
    iota3 = iota * 3
    iota32 = iota * 32
    col0 = jnp.zeros((_LANES,), jnp.int32)
    col1 = col0 + 1
    sems = (sem0, sem1)

    def phase_a(l, slot):
        res = _RES[l]
        resf = jnp.float32(res)
        dense = (res + 1) ** 3 <= _T
        base_l = l * _T

        @pl.loop(0, _GROUPS)
        def _(g):
            off = g * _LANES
            xv = xb[pl.ds(off, _LANES)]
            yv = yb[pl.ds(off, _LANES)]
            zv = zb[pl.ds(off, _LANES)]
            px = xv * resf
            py = yv * resf
            pz = zv * resf
            ix = px.astype(jnp.int32)
            iy = py.astype(jnp.int32)
            iz = pz.astype(jnp.int32)
            fx = px - ix.astype(jnp.float32)
            fy = py - iy.astype(jnp.float32)
            fz = pz - iz.astype(jnp.float32)
            ox = 1.0 - fx
            oy = 1.0 - fy
            oz = 1.0 - fz
            wxy = (ox * oy, fx * oy, ox * fy, fx * fy)

            if dense:
                s = res + 1
                ax = (ix + base_l, ix + (base_l + 1))
                ay = (iy * s, iy * s + s)
                az = (iz * (s * s), iz * (s * s) + s * s)
            else:
                xu = plsc.bitcast(ix, jnp.uint32)
                yu = plsc.bitcast(iy, jnp.uint32)
                zu = plsc.bitcast(iz, jnp.uint32)
                hx = (xu, xu + np.uint32(1))
                hy0 = yu * _P1
                hy = (hy0, hy0 + _P1)
                hz0 = zu * _P2
                hz = (hz0, hz0 + _P2)
                mask = np.uint32(_T - 1)

            row = slot * _IDX_ROWS + g
            wbase = slot * _NROWS + g * 128
            for c in range(8):
                b0, b1, b2 = c & 1, (c >> 1) & 1, (c >> 2) & 1
                if dense:
                    idx = ax[b0] + ay[b1] + az[b2]
                else:
                    h = (hx[b0] ^ hy[b1]) ^ hz[b2]
                    idx = plsc.bitcast(h & mask, jnp.int32) + base_l
                idx2[row, pl.ds(c * _LANES, _LANES)] = idx
                w = wxy[c & 3] * (fz if b2 else oz)
                w2[pl.ds(wbase + c * _LANES, _LANES)] = w

    def fire(slot):
        @pl.loop(0, _IDX_ROWS)
        def _(j):
            row = slot * _IDX_ROWS + j
            pltpu.async_copy(tbl_hbm.at[idx2.at[row]], rows2.at[row],
                             sems[slot])

    def wait_rows(slot):
        @pl.loop(0, _IDX_ROWS)
        def _(j):
            row = slot * _IDX_ROWS + j
            pltpu.make_async_copy(tbl_hbm.at[idx2.at[row]], rows2.at[row],
                                  sems[slot]).wait()

    def phase_b(l, slot):
        @pl.loop(0, _GROUPS)
        def _(g):
            rrow = slot * _IDX_ROWS + g
            rowv = col0 + rrow
            wbase = slot * _NROWS + g * 128
            acc0 = jnp.zeros((_LANES,), jnp.float32)
            acc1 = jnp.zeros((_LANES,), jnp.float32)
            for c in range(8):
                colv = iota + (c * _LANES)
                f0 = plsc.load_gather(rows2, [rowv, colv, col0])
                f1 = plsc.load_gather(rows2, [rowv, colv, col1])
                w = w2[pl.ds(wbase + c * _LANES, _LANES)]
                acc0 = acc0 + w * f0
                acc1 = acc1 + w * f1
            sidx = iota32 + (g * (_LANES * 32) + 2 * l)
            plsc.store_scatter(penc, [sidx], acc0)
            plsc.store_scatter(penc, [sidx + 1], acc1)

    @pl.loop(0, chunks)
    def _(ci):
        base = wid * ppw + ci * _P
        pltpu.sync_copy(texc_hbm.at[pl.ds(base * 3, _P * 3)], txyz)

        @pl.loop(0, _GROUPS)
        def _(g):
            gi = iota3 + g * (3 * _LANES)
            off = g * _LANES
            xb[pl.ds(off, _LANES)] = plsc.load_gather(txyz, [gi])
            yb[pl.ds(off, _LANES)] = plsc.load_gather(txyz, [gi + 1])
            zb[pl.ds(off, _LANES)] = plsc.load_gather(txyz, [gi + 2])

        phase_a(0, 0)
        fire(0)
        for l in range(1, _NUM_LEVELS):
            slot = l & 1
            phase_a(l, slot)
            fire(slot)
            wait_rows(1 - slot)
            phase_b(l - 1, 1 - slot)
        wait_rows(1)
        phase_b(_NUM_LEVELS - 1, 1)

        pltpu.sync_copy(penc, out_hbm.at[pl.ds(base * 32, _P * 32)])


@functools.lru_cache(maxsize=None)
def _build_encode(n_pts):
    assert n_pts % (_NW * _P) == 0
    mesh = plsc.VectorSubcoreMesh(core_axis_name="c", subcore_axis_name="s")
    return pl.kernel(
        _encode_body,
        out_type=jax.ShapeDtypeStruct((n_pts * 32,), jnp.float32),
        mesh=mesh,
        compiler_params=pltpu.CompilerParams(needs_layout_passes=False,
                                             use_tc_tiling_on_sc=False),
        scratch_types=[
            pltpu.VMEM((_P * 3,), jnp.float32),
            pltpu.VMEM((_P,), jnp.float32),
            pltpu.VMEM((_P,), jnp.float32),
            pltpu.VMEM((_P,), jnp.float32),
            pltpu.VMEM((2 * _IDX_ROWS, 128), jnp.int32),
            pltpu.VMEM((2 * _NROWS,), jnp.float32),
            pltpu.VMEM((2 * _IDX_ROWS, 128, _FEAT), jnp.float32),
            pltpu.VMEM((_P * 32,), jnp.float32),
            pltpu.SemaphoreType.DMA,
            pltpu.SemaphoreType.DMA,
        ],
    )


def _mlp_body(x_ref, w0_ref, w1_ref, w2_ref, o_ref):
    x = x_ref[...]
    h = jnp.maximum(jnp.dot(x, w0_ref[...], preferred_element_type=jnp.float32), 0.0)
    h = jnp.maximum(jnp.dot(h, w1_ref[...], preferred_element_type=jnp.float32), 0.0)
    o_ref[...] = jnp.dot(h, w2_ref[...], preferred_element_type=jnp.float32)


@functools.lru_cache(maxsize=None)
def _build_mlp(n_pts):
    blk = 8192
    assert n_pts % blk == 0
    return pl.pallas_call(
        _mlp_body,
        grid=(n_pts // blk,),
        in_specs=[
            pl.BlockSpec((blk, 32), lambda i: (i, 0)),
            pl.BlockSpec((32, 32), lambda i: (0, 0)),
            pl.BlockSpec((32, 32), lambda i: (0, 0)),
            pl.BlockSpec((32, 4), lambda i: (0, 0)),
        ],
        out_specs=pl.BlockSpec((blk, 4), lambda i: (i, 0)),
        out_shape=jax.ShapeDtypeStruct((n_pts, 4), jnp.float32),
    )


def kernel(texc, table, W0, W1, W2):
    x = texc.reshape(-1, 3).astype(jnp.float32)
    n_pts = x.shape[0]
    tbl = table.reshape(_NUM_LEVELS * _T, _FEAT)
    p_enc = _build_encode(n_pts)(x.reshape(-1), tbl).reshape(n_pts, 32)
    return _build_mlp(n_pts)(p_enc, W0.T, W1.T, W2.T)


# speedup vs baseline: 1.2063x; 1.2063x over previous
import functools

import numpy as np
import jax
import jax.numpy as jnp
from jax import lax
from jax.experimental import pallas as pl
from jax.experimental.pallas import tpu as pltpu
from jax.experimental.pallas import tpu_sc as plsc

_NUM_LEVELS = 16
_FEAT = 2
_T = 1 << 19
_BASE_RES = 16
_SCALE = float(np.exp(np.log(4096.0 / 16.0) / (_NUM_LEVELS - 1)))
_RES = [int(np.floor(_BASE_RES * _SCALE ** l)) for l in range(_NUM_LEVELS)]
_P1 = np.uint32(2654435761)
_P2 = np.uint32(805459861)

_NC = 2
_NS = 16
_NW = _NC * _NS
_LANES = 16

_P = 512
_GROUPS = _P // _LANES
_NROWS = _P * 8
_IDX_ROWS = _NROWS // 128


def _encode_body(texc_hbm, tbl_hbm, out_hbm, txyz, xb, yb, zb, idx2, w2,
                 rem2, rows2, penc, sem0, sem1):
    n_pts = texc_hbm.shape[0] // 3
    ppw = n_pts // _NW
    chunks = ppw // _P

    cid = lax.axis_index("c")
    sid = lax.axis_index("s")
    wid = sid * _NC + cid

    iota = lax.iota(jnp.int32, _LANES)
    iota3 = iota * 3
    iota32 = iota * 32
    col0 = jnp.zeros((_LANES,), jnp.int32)
    sems = (sem0, sem1)

    def phase_a(l, slot):
        res = _RES[l]
        resf = jnp.float32(res)
        dense = (res + 1) ** 3 <= _T
        base_l = l * _T

        @pl.loop(0, _GROUPS)
        def _(g):
            off = g * _LANES
            xv = xb[pl.ds(off, _LANES)]
            yv = yb[pl.ds(off, _LANES)]
            zv = zb[pl.ds(off, _LANES)]
            px = xv * resf
            py = yv * resf
            pz = zv * resf
            ix = px.astype(jnp.int32)
            iy = py.astype(jnp.int32)
            iz = pz.astype(jnp.int32)
            fx = px - ix.astype(jnp.float32)
            fy = py - iy.astype(jnp.float32)
            fz = pz - iz.astype(jnp.float32)
            ox = 1.0 - fx
            oy = 1.0 - fy
            oz = 1.0 - fz
            wxy = (ox * oy, fx * oy, ox * fy, fx * fy)

            if dense:
                s = res + 1
                ax = (ix + base_l, ix + (base_l + 1))
                ay = (iy * s, iy * s + s)
                az = (iz * (s * s), iz * (s * s) + s * s)
            else:
                xu = plsc.bitcast(ix, jnp.uint32)
                yu = plsc.bitcast(iy, jnp.uint32)
                zu = plsc.bitcast(iz, jnp.uint32)
                hx = (xu, xu + np.uint32(1))
                hy0 = yu * _P1
                hy = (hy0, hy0 + _P1)
                hz0 = zu * _P2
                hz = (hz0, hz0 + _P2)
                mask = np.uint32(_T - 1)

            row = slot * _IDX_ROWS + g
            wbase = slot * _NROWS + g * 128
            for c in range(8):
                b0, b1, b2 = c & 1, (c >> 1) & 1, (c >> 2) & 1
                if dense:
                    idx = ax[b0] + ay[b1] + az[b2]
                else:
                    h = (hx[b0] ^ hy[b1]) ^ hz[b2]
                    idx = plsc.bitcast(h & mask, jnp.int32) + base_l
                idx2[row, pl.ds(c * _LANES, _LANES)] = idx >> 2
                rem2[pl.ds(wbase + c * _LANES, _LANES)] = (idx & 3) << 1
                w = wxy[c & 3] * (fz if b2 else oz)
                w2[pl.ds(wbase + c * _LANES, _LANES)] = w

    def fire(slot):
        @pl.loop(0, _IDX_ROWS)
        def _(j):
            row = slot * _IDX_ROWS + j
            pltpu.async_copy(tbl_hbm.at[idx2.at[row]], rows2.at[row],
                             sems[slot])

    def wait_rows(slot):
        @pl.loop(0, _IDX_ROWS)
        def _(j):
            row = slot * _IDX_ROWS + j
            pltpu.make_async_copy(tbl_hbm.at[idx2.at[row]], rows2.at[row],
                                  sems[slot]).wait()

    def phase_b(l, slot):
        @pl.loop(0, _GROUPS)
        def _(g):
            rrow = slot * _IDX_ROWS + g
            rowv = col0 + rrow
            wbase = slot * _NROWS + g * 128
            acc0 = jnp.zeros((_LANES,), jnp.float32)
            acc1 = jnp.zeros((_LANES,), jnp.float32)
            for c in range(8):
                colv = iota + (c * _LANES)
                wordv = rem2[pl.ds(wbase + c * _LANES, _LANES)]
                f0 = plsc.load_gather(rows2, [rowv, colv, wordv])
                f1 = plsc.load_gather(rows2, [rowv, colv, wordv + 1])
                w = w2[pl.ds(wbase + c * _LANES, _LANES)]
                acc0 = acc0 + w * f0
                acc1 = acc1 + w * f1
            sidx = iota32 + (g * (_LANES * 32) + 2 * l)
            plsc.store_scatter(penc, [sidx], acc0)
            plsc.store_scatter(penc, [sidx + 1], acc1)

    @pl.loop(0, chunks)
    def _(ci):
        base = wid * ppw + ci * _P
        pltpu.sync_copy(texc_hbm.at[pl.ds(base * 3, _P * 3)], txyz)

        @pl.loop(0, _GROUPS)
        def _(g):
            gi = iota3 + g * (3 * _LANES)
            off = g * _LANES
            xb[pl.ds(off, _LANES)] = plsc.load_gather(txyz, [gi])
            yb[pl.ds(off, _LANES)] = plsc.load_gather(txyz, [gi + 1])
            zb[pl.ds(off, _LANES)] = plsc.load_gather(txyz, [gi + 2])

        phase_a(0, 0)
        fire(0)
        for l in range(1, _NUM_LEVELS):
            slot = l & 1
            phase_a(l, slot)
            fire(slot)
            wait_rows(1 - slot)
            phase_b(l - 1, 1 - slot)
        wait_rows(1)
        phase_b(_NUM_LEVELS - 1, 1)

        pltpu.sync_copy(penc, out_hbm.at[pl.ds(base * 32, _P * 32)])


@functools.lru_cache(maxsize=None)
def _build_encode(n_pts):
    assert n_pts % (_NW * _P) == 0
    mesh = plsc.VectorSubcoreMesh(core_axis_name="c", subcore_axis_name="s")
    return pl.kernel(
        _encode_body,
        out_type=jax.ShapeDtypeStruct((n_pts * 32,), jnp.float32),
        mesh=mesh,
        compiler_params=pltpu.CompilerParams(needs_layout_passes=False,
                                             use_tc_tiling_on_sc=False),
        scratch_types=[
            pltpu.VMEM((_P * 3,), jnp.float32),
            pltpu.VMEM((_P,), jnp.float32),
            pltpu.VMEM((_P,), jnp.float32),
            pltpu.VMEM((_P,), jnp.float32),
            pltpu.VMEM((2 * _IDX_ROWS, 128), jnp.int32),
            pltpu.VMEM((2 * _NROWS,), jnp.float32),
            pltpu.VMEM((2 * _NROWS,), jnp.int32),
            pltpu.VMEM((2 * _IDX_ROWS, 128, 8), jnp.float32),
            pltpu.VMEM((_P * 32,), jnp.float32),
            pltpu.SemaphoreType.DMA,
            pltpu.SemaphoreType.DMA,
        ],
    )


def _mlp_body(x_ref, w0_ref, w1_ref, w2_ref, o_ref):
    x = x_ref[...]
    h = jnp.maximum(jnp.dot(x, w0_ref[...], preferred_element_type=jnp.float32), 0.0)
    h = jnp.maximum(jnp.dot(h, w1_ref[...], preferred_element_type=jnp.float32), 0.0)
    o_ref[...] = jnp.dot(h, w2_ref[...], preferred_element_type=jnp.float32)


@functools.lru_cache(maxsize=None)
def _build_mlp(n_pts):
    blk = 8192
    assert n_pts % blk == 0
    return pl.pallas_call(
        _mlp_body,
        grid=(n_pts // blk,),
        in_specs=[
            pl.BlockSpec((blk, 32), lambda i: (i, 0)),
            pl.BlockSpec((32, 32), lambda i: (0, 0)),
            pl.BlockSpec((32, 32), lambda i: (0, 0)),
            pl.BlockSpec((32, 4), lambda i: (0, 0)),
        ],
        out_specs=pl.BlockSpec((blk, 4), lambda i: (i, 0)),
        out_shape=jax.ShapeDtypeStruct((n_pts, 4), jnp.float32),
    )


def kernel(texc, table, W0, W1, W2):
    x = texc.reshape(-1, 3).astype(jnp.float32)
    n_pts = x.shape[0]
    tbl = table.reshape(_NUM_LEVELS * _T * _FEAT // 8, 8)
    p_enc = _build_encode(n_pts)(x.reshape(-1), tbl).reshape(n_pts, 32)
    return _build_mlp(n_pts)(p_enc, W0.T, W1.T, W2.T)


# speedup vs baseline: 4.7862x; 3.9677x over previous
import functools

import numpy as np
import jax
import jax.numpy as jnp
from jax import lax
from jax.experimental import pallas as pl
from jax.experimental.pallas import tpu as pltpu
from jax.experimental.pallas import tpu_sc as plsc

_NUM_LEVELS = 16
_FEAT = 2
_T = 1 << 19
_BASE_RES = 16
_SCALE = float(np.exp(np.log(4096.0 / 16.0) / (_NUM_LEVELS - 1)))
_RES = [int(np.floor(_BASE_RES * _SCALE ** l)) for l in range(_NUM_LEVELS)]
_P1 = np.uint32(2654435761)
_P2 = np.uint32(805459861)

_NC = 2
_NS = 16
_NW = _NC * _NS
_LANES = 16

_P = 256
_GROUPS = _P // _LANES
_NROWS = _P * 8
_IDX_ROWS = _NROWS // 128
_TROWS = _T // 8


def _encode_body(texc_hbm, p0_hbm, p1_hbm, out_hbm, txyz, xb, yb, zb, idx2,
                 w2, rem2, rows0, rows1, penc, sem0, sem1):
    n_pts = texc_hbm.shape[0] // 3
    ppw = n_pts // _NW
    chunks = ppw // _P

    cid = lax.axis_index("c")
    sid = lax.axis_index("s")
    wid = sid * _NC + cid

    iota = lax.iota(jnp.int32, _LANES)
    iota3 = iota * 3
    iota32 = iota * 32
    col0 = jnp.zeros((_LANES,), jnp.int32)
    sems = (sem0, sem1)

    def phase_a(l, slot):
        res = _RES[l]
        resf = jnp.float32(res)
        dense = (res + 1) ** 3 <= _T
        rbase = l * _TROWS

        @pl.loop(0, _GROUPS)
        def _(g):
            off = g * _LANES
            xv = xb[pl.ds(off, _LANES)]
            yv = yb[pl.ds(off, _LANES)]
            zv = zb[pl.ds(off, _LANES)]
            px = xv * resf
            py = yv * resf
            pz = zv * resf
            ix = px.astype(jnp.int32)
            iy = py.astype(jnp.int32)
            iz = pz.astype(jnp.int32)
            fx = px - ix.astype(jnp.float32)
            fy = py - iy.astype(jnp.float32)
            fz = pz - iz.astype(jnp.float32)
            ox = 1.0 - fx
            oy = 1.0 - fy
            oz = 1.0 - fz
            wxy = (ox * oy, fx * oy, ox * fy, fx * fy)

            if dense:
                s = res + 1
                ax = (ix, ix + 1)
                ay = (iy * s, iy * s + s)
                az = (iz * (s * s), iz * (s * s) + s * s)
            else:
                xu = plsc.bitcast(ix, jnp.uint32)
                yu = plsc.bitcast(iy, jnp.uint32)
                zu = plsc.bitcast(iz, jnp.uint32)
                hx = (xu, xu + np.uint32(1))
                hy0 = yu * _P1
                hy = (hy0, hy0 + _P1)
                hz0 = zu * _P2
                hz = (hz0, hz0 + _P2)
                mask = np.uint32(_T - 1)

            row = slot * _IDX_ROWS + g
            wbase = slot * _NROWS + g * 128
            for c in range(8):
                b0, b1, b2 = c & 1, (c >> 1) & 1, (c >> 2) & 1
                if dense:
                    idx = ax[b0] + ay[b1] + az[b2]
                else:
                    h = (hx[b0] ^ hy[b1]) ^ hz[b2]
                    idx = plsc.bitcast(h & mask, jnp.int32)
                idx2[row, pl.ds(c * _LANES, _LANES)] = rbase + (idx >> 3)
                rem2[pl.ds(wbase + c * _LANES, _LANES)] = idx & 7
                w = wxy[c & 3] * (fz if b2 else oz)
                w2[pl.ds(wbase + c * _LANES, _LANES)] = w

    def fire(slot):
        @pl.loop(0, _IDX_ROWS)
        def _(j):
            row = slot * _IDX_ROWS + j
            pltpu.async_copy(p0_hbm.at[idx2.at[row]], rows0.at[row],
                             sems[slot])
            pltpu.async_copy(p1_hbm.at[idx2.at[row]], rows1.at[row],
                             sems[slot])

    def wait_rows(slot):
        @pl.loop(0, _IDX_ROWS)
        def _(j):
            row = slot * _IDX_ROWS + j
            pltpu.make_async_copy(p0_hbm.at[idx2.at[row]], rows0.at[row],
                                  sems[slot]).wait()
            pltpu.make_async_copy(p1_hbm.at[idx2.at[row]], rows1.at[row],
                                  sems[slot]).wait()

    def phase_b(l, slot):
        @pl.loop(0, _GROUPS)
        def _(g):
            rrow = slot * _IDX_ROWS + g
            rowv = col0 + rrow
            wbase = slot * _NROWS + g * 128
            acc0 = jnp.zeros((_LANES,), jnp.float32)
            acc1 = jnp.zeros((_LANES,), jnp.float32)
            for c in range(8):
                colv = iota + (c * _LANES)
                wordv = rem2[pl.ds(wbase + c * _LANES, _LANES)]
                f0 = plsc.load_gather(rows0, [rowv, colv, wordv])
                f1 = plsc.load_gather(rows1, [rowv, colv, wordv])
                w = w2[pl.ds(wbase + c * _LANES, _LANES)]
                acc0 = acc0 + w * f0
                acc1 = acc1 + w * f1
            sidx = iota32 + (g * (_LANES * 32) + 2 * l)
            plsc.store_scatter(penc, [sidx], acc0)
            plsc.store_scatter(penc, [sidx + 1], acc1)

    @pl.loop(0, chunks)
    def _(ci):
        base = wid * ppw + ci * _P
        pltpu.sync_copy(texc_hbm.at[pl.ds(base * 3, _P * 3)], txyz)

        @pl.loop(0, _GROUPS)
        def _(g):
            gi = iota3 + g * (3 * _LANES)
            off = g * _LANES
            xb[pl.ds(off, _LANES)] = plsc.load_gather(txyz, [gi])
            yb[pl.ds(off, _LANES)] = plsc.load_gather(txyz, [gi + 1])
            zb[pl.ds(off, _LANES)] = plsc.load_gather(txyz, [gi + 2])

        phase_a(0, 0)
        fire(0)
        for l in range(1, _NUM_LEVELS):
            slot = l & 1
            phase_a(l, slot)
            fire(slot)
            wait_rows(1 - slot)
            phase_b(l - 1, 1 - slot)
        wait_rows(1)
        phase_b(_NUM_LEVELS - 1, 1)

        pltpu.sync_copy(penc, out_hbm.at[pl.ds(base * 32, _P * 32)])


@functools.lru_cache(maxsize=None)
def _build_encode(n_pts):
    assert n_pts % (_NW * _P) == 0
    mesh = plsc.VectorSubcoreMesh(core_axis_name="c", subcore_axis_name="s")
    return pl.kernel(
        _encode_body,
        out_type=jax.ShapeDtypeStruct((n_pts * 32,), jnp.float32),
        mesh=mesh,
        compiler_params=pltpu.CompilerParams(needs_layout_passes=False,
                                             use_tc_tiling_on_sc=False),
        scratch_types=[
            pltpu.VMEM((_P * 3,), jnp.float32),
            pltpu.VMEM((_P,), jnp.float32),
            pltpu.VMEM((_P,), jnp.float32),
            pltpu.VMEM((_P,), jnp.float32),
            pltpu.VMEM((2 * _IDX_ROWS, 128), jnp.int32),
            pltpu.VMEM((2 * _NROWS,), jnp.float32),
            pltpu.VMEM((2 * _NROWS,), jnp.int32),
            pltpu.VMEM((2 * _IDX_ROWS, 128, 8), jnp.float32),
            pltpu.VMEM((2 * _IDX_ROWS, 128, 8), jnp.float32),
            pltpu.VMEM((_P * 32,), jnp.float32),
            pltpu.SemaphoreType.DMA,
            pltpu.SemaphoreType.DMA,
        ],
    )


def _mlp_body(x_ref, w0_ref, w1_ref, w2_ref, o_ref):
    x = x_ref[...]
    h = jnp.maximum(jnp.dot(x, w0_ref[...], preferred_element_type=jnp.float32), 0.0)
    h = jnp.maximum(jnp.dot(h, w1_ref[...], preferred_element_type=jnp.float32), 0.0)
    o_ref[...] = jnp.dot(h, w2_ref[...], preferred_element_type=jnp.float32)


@functools.lru_cache(maxsize=None)
def _build_mlp(n_pts):
    blk = 8192
    assert n_pts % blk == 0
    return pl.pallas_call(
        _mlp_body,
        grid=(n_pts // blk,),
        in_specs=[
            pl.BlockSpec((blk, 32), lambda i: (i, 0)),
            pl.BlockSpec((32, 32), lambda i: (0, 0)),
            pl.BlockSpec((32, 32), lambda i: (0, 0)),
            pl.BlockSpec((32, 4), lambda i: (0, 0)),
        ],
        out_specs=pl.BlockSpec((blk, 4), lambda i: (i, 0)),
        out_shape=jax.ShapeDtypeStruct((n_pts, 4), jnp.float32),
    )


def kernel(texc, table, W0, W1, W2):
    x = texc.reshape(-1, 3).astype(jnp.float32)
    n_pts = x.shape[0]
    p0 = table[:, :, 0].reshape(_NUM_LEVELS * _TROWS, 8)
    p1 = table[:, :, 1].reshape(_NUM_LEVELS * _TROWS, 8)
    p_enc = _build_encode(n_pts)(x.reshape(-1), p0, p1).reshape(n_pts, 32)
    return _build_mlp(n_pts)(p_enc, W0.T, W1.T, W2.T)


# speedup vs baseline: 8.1343x; 1.6995x over previous
import functools

import numpy as np
import jax
import jax.numpy as jnp
from jax import lax
from jax.experimental import pallas as pl
from jax.experimental.pallas import tpu as pltpu
from jax.experimental.pallas import tpu_sc as plsc

_NUM_LEVELS = 16
_FEAT = 2
_T = 1 << 19
_BASE_RES = 16
_SCALE = float(np.exp(np.log(4096.0 / 16.0) / (_NUM_LEVELS - 1)))
_RES = [int(np.floor(_BASE_RES * _SCALE ** l)) for l in range(_NUM_LEVELS)]
_P1 = np.uint32(2654435761)
_P2 = np.uint32(805459861)

_NC = 2
_NS = 16
_NW = _NC * _NS
_LANES = 16

_ENTRIES = _NUM_LEVELS * _T
_P = 512
_GROUPS = _P // _LANES
_NROWS = _P * 8
_IDX_ROWS = _NROWS // 128

_RP_CH = 4096


def _repack_body(p0_hbm, p1_hbm, out_hbm, in0, in1, outb):
    epw = _ENTRIES // _NW
    chunks = epw // _RP_CH

    cid = lax.axis_index("c")
    sid = lax.axis_index("s")
    wid = sid * _NC + cid

    iota = lax.iota(jnp.int32, _LANES)
    iota2 = iota * 2

    @pl.loop(0, chunks)
    def _(ci):
        ebase = wid * epw + ci * _RP_CH
        pltpu.sync_copy(p0_hbm.at[pl.ds(ebase, _RP_CH)], in0)
        pltpu.sync_copy(p1_hbm.at[pl.ds(ebase, _RP_CH)], in1)

        @pl.loop(0, _RP_CH // _LANES)
        def _(g):
            v0 = in0[pl.ds(g * _LANES, _LANES)]
            v1 = in1[pl.ds(g * _LANES, _LANES)]
            base = g * (2 * _LANES)
            plsc.store_scatter(outb, [iota2 + base], v0)
            plsc.store_scatter(outb, [iota2 + (base + 1)], v1)

        pltpu.sync_copy(outb, out_hbm.at[pl.ds(ebase * 2, _RP_CH * 2)])


@functools.lru_cache(maxsize=None)
def _build_repack():
    mesh = plsc.VectorSubcoreMesh(core_axis_name="c", subcore_axis_name="s")
    return pl.kernel(
        _repack_body,
        out_type=jax.ShapeDtypeStruct((_ENTRIES * 2,), jnp.float32),
        mesh=mesh,
        compiler_params=pltpu.CompilerParams(needs_layout_passes=False,
                                             use_tc_tiling_on_sc=False),
        scratch_types=[
            pltpu.VMEM((_RP_CH,), jnp.float32),
            pltpu.VMEM((_RP_CH,), jnp.float32),
            pltpu.VMEM((_RP_CH * 2,), jnp.float32),
        ],
    )


def _encode_body(texc_hbm, tbl_hbm, out_hbm, txyz, xb, yb, zb, idx2, w2,
                 rem2, rows2, penc, sem0, sem1):
    n_pts = texc_hbm.shape[0] // 3
    ppw = n_pts // _NW
    chunks = ppw // _P

    cid = lax.axis_index("c")
    sid = lax.axis_index("s")
    wid = sid * _NC + cid

    iota = lax.iota(jnp.int32, _LANES)
    iota3 = iota * 3
    iota32 = iota * 32
    col0 = jnp.zeros((_LANES,), jnp.int32)
    sems = (sem0, sem1)

    def phase_a(l, slot):
        res = _RES[l]
        resf = jnp.float32(res)
        dense = (res + 1) ** 3 <= _T
        base_l = l * _T

        @pl.loop(0, _GROUPS)
        def _(g):
            off = g * _LANES
            xv = xb[pl.ds(off, _LANES)]
            yv = yb[pl.ds(off, _LANES)]
            zv = zb[pl.ds(off, _LANES)]
            px = xv * resf
            py = yv * resf
            pz = zv * resf
            ix = px.astype(jnp.int32)
            iy = py.astype(jnp.int32)
            iz = pz.astype(jnp.int32)
            fx = px - ix.astype(jnp.float32)
            fy = py - iy.astype(jnp.float32)
            fz = pz - iz.astype(jnp.float32)
            ox = 1.0 - fx
            oy = 1.0 - fy
            oz = 1.0 - fz
            wxy = (ox * oy, fx * oy, ox * fy, fx * fy)

            if dense:
                s = res + 1
                ax = (ix + base_l, ix + (base_l + 1))
                ay = (iy * s, iy * s + s)
                az = (iz * (s * s), iz * (s * s) + s * s)
            else:
                xu = plsc.bitcast(ix, jnp.uint32)
                yu = plsc.bitcast(iy, jnp.uint32)
                zu = plsc.bitcast(iz, jnp.uint32)
                hx = (xu, xu + np.uint32(1))
                hy0 = yu * _P1
                hy = (hy0, hy0 + _P1)
                hz0 = zu * _P2
                hz = (hz0, hz0 + _P2)
                mask = np.uint32(_T - 1)

            row = slot * _IDX_ROWS + g
            wbase = slot * _NROWS + g * 128
            for c in range(8):
                b0, b1, b2 = c & 1, (c >> 1) & 1, (c >> 2) & 1
                if dense:
                    idx = ax[b0] + ay[b1] + az[b2]
                else:
                    h = (hx[b0] ^ hy[b1]) ^ hz[b2]
                    idx = plsc.bitcast(h & mask, jnp.int32) + base_l
                idx2[row, pl.ds(c * _LANES, _LANES)] = idx >> 2
                rem2[pl.ds(wbase + c * _LANES, _LANES)] = (idx & 3) << 1
                w = wxy[c & 3] * (fz if b2 else oz)
                w2[pl.ds(wbase + c * _LANES, _LANES)] = w

    def fire(slot):
        @pl.loop(0, _IDX_ROWS)
        def _(j):
            row = slot * _IDX_ROWS + j
            pltpu.async_copy(tbl_hbm.at[idx2.at[row]], rows2.at[row],
                             sems[slot])

    def wait_rows(slot):
        @pl.loop(0, _IDX_ROWS)
        def _(j):
            row = slot * _IDX_ROWS + j
            pltpu.make_async_copy(tbl_hbm.at[idx2.at[row]], rows2.at[row],
                                  sems[slot]).wait()

    def phase_b(l, slot):
        @pl.loop(0, _GROUPS)
        def _(g):
            rrow = slot * _IDX_ROWS + g
            rowv = col0 + rrow
            wbase = slot * _NROWS + g * 128
            acc0 = jnp.zeros((_LANES,), jnp.float32)
            acc1 = jnp.zeros((_LANES,), jnp.float32)
            for c in range(8):
                colv = iota + (c * _LANES)
                wordv = rem2[pl.ds(wbase + c * _LANES, _LANES)]
                f0 = plsc.load_gather(rows2, [rowv, colv, wordv])
                f1 = plsc.load_gather(rows2, [rowv, colv, wordv + 1])
                w = w2[pl.ds(wbase + c * _LANES, _LANES)]
                acc0 = acc0 + w * f0
                acc1 = acc1 + w * f1
            sidx = iota32 + (g * (_LANES * 32) + 2 * l)
            plsc.store_scatter(penc, [sidx], acc0)
            plsc.store_scatter(penc, [sidx + 1], acc1)

    @pl.loop(0, chunks)
    def _(ci):
        base = wid * ppw + ci * _P
        pltpu.sync_copy(texc_hbm.at[pl.ds(base * 3, _P * 3)], txyz)

        @pl.loop(0, _GROUPS)
        def _(g):
            gi = iota3 + g * (3 * _LANES)
            off = g * _LANES
            xb[pl.ds(off, _LANES)] = plsc.load_gather(txyz, [gi])
            yb[pl.ds(off, _LANES)] = plsc.load_gather(txyz, [gi + 1])
            zb[pl.ds(off, _LANES)] = plsc.load_gather(txyz, [gi + 2])

        phase_a(0, 0)
        fire(0)
        for l in range(1, _NUM_LEVELS):
            slot = l & 1
            phase_a(l, slot)
            fire(slot)
            wait_rows(1 - slot)
            phase_b(l - 1, 1 - slot)
        wait_rows(1)
        phase_b(_NUM_LEVELS - 1, 1)

        pltpu.sync_copy(penc, out_hbm.at[pl.ds(base * 32, _P * 32)])


@functools.lru_cache(maxsize=None)
def _build_encode(n_pts):
    assert n_pts % (_NW * _P) == 0
    mesh = plsc.VectorSubcoreMesh(core_axis_name="c", subcore_axis_name="s")
    return pl.kernel(
        _encode_body,
        out_type=jax.ShapeDtypeStruct((n_pts * 32,), jnp.float32),
        mesh=mesh,
        compiler_params=pltpu.CompilerParams(needs_layout_passes=False,
                                             use_tc_tiling_on_sc=False),
        scratch_types=[
            pltpu.VMEM((_P * 3,), jnp.float32),
            pltpu.VMEM((_P,), jnp.float32),
            pltpu.VMEM((_P,), jnp.float32),
            pltpu.VMEM((_P,), jnp.float32),
            pltpu.VMEM((2 * _IDX_ROWS, 128), jnp.int32),
            pltpu.VMEM((2 * _NROWS,), jnp.float32),
            pltpu.VMEM((2 * _NROWS,), jnp.int32),
            pltpu.VMEM((2 * _IDX_ROWS, 128, 8), jnp.float32),
            pltpu.VMEM((_P * 32,), jnp.float32),
            pltpu.SemaphoreType.DMA,
            pltpu.SemaphoreType.DMA,
        ],
    )


def _mlp_body(x_ref, w0_ref, w1_ref, w2_ref, o_ref):
    x = x_ref[...]
    h = jnp.maximum(jnp.dot(x, w0_ref[...], preferred_element_type=jnp.float32), 0.0)
    h = jnp.maximum(jnp.dot(h, w1_ref[...], preferred_element_type=jnp.float32), 0.0)
    o_ref[...] = jnp.dot(h, w2_ref[...], preferred_element_type=jnp.float32)


@functools.lru_cache(maxsize=None)
def _build_mlp(n_pts):
    blk = 8192
    assert n_pts % blk == 0
    return pl.pallas_call(
        _mlp_body,
        grid=(n_pts // blk,),
        in_specs=[
            pl.BlockSpec((blk, 32), lambda i: (i, 0)),
            pl.BlockSpec((32, 32), lambda i: (0, 0)),
            pl.BlockSpec((32, 32), lambda i: (0, 0)),
            pl.BlockSpec((32, 4), lambda i: (0, 0)),
        ],
        out_specs=pl.BlockSpec((blk, 4), lambda i: (i, 0)),
        out_shape=jax.ShapeDtypeStruct((n_pts, 4), jnp.float32),
    )


def kernel(texc, table, W0, W1, W2):
    x = texc.reshape(-1, 3).astype(jnp.float32)
    n_pts = x.shape[0]
    p0 = table[:, :, 0].reshape(-1)
    p1 = table[:, :, 1].reshape(-1)
    tbl_i = _build_repack()(p0, p1).reshape(_ENTRIES // 4, 8)
    p_enc = _build_encode(n_pts)(x.reshape(-1), tbl_i).reshape(n_pts, 32)
    return _build_mlp(n_pts)(p_enc, W0.T, W1.T, W2.T)


# speedup vs baseline: 8.3212x; 1.0230x over previous
import functools

import numpy as np
import jax
import jax.numpy as jnp
from jax import lax
from jax.experimental import pallas as pl
from jax.experimental.pallas import tpu as pltpu
from jax.experimental.pallas import tpu_sc as plsc

_NUM_LEVELS = 16
_FEAT = 2
_T = 1 << 19
_BASE_RES = 16
_SCALE = float(np.exp(np.log(4096.0 / 16.0) / (_NUM_LEVELS - 1)))
_RES = [int(np.floor(_BASE_RES * _SCALE ** l)) for l in range(_NUM_LEVELS)]
_P1 = np.uint32(2654435761)
_P2 = np.uint32(805459861)

_NC = 2
_NS = 16
_NW = _NC * _NS
_LANES = 16

_ENTRIES = _NUM_LEVELS * _T
_P = 512
_GROUPS = _P // _LANES
_NROWS = _P * 8
_IDX_ROWS = _NROWS // 128

_RP_CH = 16384


def _repack_body(p0_hbm, p1_hbm, out_hbm, in0, in1, outb):
    epw = _ENTRIES // _NW
    chunks = epw // _RP_CH

    cid = lax.axis_index("c")
    sid = lax.axis_index("s")
    wid = sid * _NC + cid

    iota = lax.iota(jnp.int32, _LANES)
    iota2 = iota * 2

    @pl.loop(0, chunks)
    def _(ci):
        ebase = wid * epw + ci * _RP_CH
        pltpu.sync_copy(p0_hbm.at[pl.ds(ebase, _RP_CH)], in0)
        pltpu.sync_copy(p1_hbm.at[pl.ds(ebase, _RP_CH)], in1)

        @pl.loop(0, _RP_CH // _LANES)
        def _(g):
            v0 = in0[pl.ds(g * _LANES, _LANES)]
            v1 = in1[pl.ds(g * _LANES, _LANES)]
            base = g * (2 * _LANES)
            plsc.store_scatter(outb, [iota2 + base], v0)
            plsc.store_scatter(outb, [iota2 + (base + 1)], v1)

        pltpu.sync_copy(outb, out_hbm.at[pl.ds(ebase * 2, _RP_CH * 2)])


@functools.lru_cache(maxsize=None)
def _build_repack():
    mesh = plsc.VectorSubcoreMesh(core_axis_name="c", subcore_axis_name="s")
    return pl.kernel(
        _repack_body,
        out_type=jax.ShapeDtypeStruct((_ENTRIES * 2,), jnp.float32),
        mesh=mesh,
        compiler_params=pltpu.CompilerParams(needs_layout_passes=False,
                                             use_tc_tiling_on_sc=False),
        scratch_types=[
            pltpu.VMEM((_RP_CH,), jnp.float32),
            pltpu.VMEM((_RP_CH,), jnp.float32),
            pltpu.VMEM((_RP_CH * 2,), jnp.float32),
        ],
    )


def _encode_body(texc_hbm, tbl_hbm, out_hbm, txyz, xb, yb, zb, idx2, w2,
                 rem2, rows2, penc, sem0, sem1):
    n_pts = texc_hbm.shape[0] // 3
    ppw = n_pts // _NW
    chunks = ppw // _P

    cid = lax.axis_index("c")
    sid = lax.axis_index("s")
    wid = sid * _NC + cid

    iota = lax.iota(jnp.int32, _LANES)
    iota3 = iota * 3
    iota32 = iota * 32
    col0 = jnp.zeros((_LANES,), jnp.int32)
    sems = (sem0, sem1)

    def phase_a(l, slot):
        res = _RES[l]
        resf = jnp.float32(res)
        dense = (res + 1) ** 3 <= _T
        base_l = l * _T

        @pl.loop(0, _GROUPS)
        def _(g):
            off = g * _LANES
            xv = xb[pl.ds(off, _LANES)]
            yv = yb[pl.ds(off, _LANES)]
            zv = zb[pl.ds(off, _LANES)]
            px = xv * resf
            py = yv * resf
            pz = zv * resf
            ix = px.astype(jnp.int32)
            iy = py.astype(jnp.int32)
            iz = pz.astype(jnp.int32)
            fx = px - ix.astype(jnp.float32)
            fy = py - iy.astype(jnp.float32)
            fz = pz - iz.astype(jnp.float32)
            ox = 1.0 - fx
            oy = 1.0 - fy
            oz = 1.0 - fz
            wxy = (ox * oy, fx * oy, ox * fy, fx * fy)

            if dense:
                s = res + 1
                ax = (ix + base_l, ix + (base_l + 1))
                ay = (iy * s, iy * s + s)
                az = (iz * (s * s), iz * (s * s) + s * s)
            else:
                xu = plsc.bitcast(ix, jnp.uint32)
                yu = plsc.bitcast(iy, jnp.uint32)
                zu = plsc.bitcast(iz, jnp.uint32)
                hx = (xu, xu + np.uint32(1))
                hy0 = yu * _P1
                hy = (hy0, hy0 + _P1)
                hz0 = zu * _P2
                hz = (hz0, hz0 + _P2)
                mask = np.uint32(_T - 1)

            row = slot * _IDX_ROWS + g
            wbase = slot * _NROWS + g * 128
            for c in range(8):
                b0, b1, b2 = c & 1, (c >> 1) & 1, (c >> 2) & 1
                if dense:
                    idx = ax[b0] + ay[b1] + az[b2]
                else:
                    h = (hx[b0] ^ hy[b1]) ^ hz[b2]
                    idx = plsc.bitcast(h & mask, jnp.int32) + base_l
                idx2[row, pl.ds(c * _LANES, _LANES)] = idx >> 2
                rem2[pl.ds(wbase + c * _LANES, _LANES)] = (idx & 3) << 1
                w = wxy[c & 3] * (fz if b2 else oz)
                w2[pl.ds(wbase + c * _LANES, _LANES)] = w

    def fire(slot):
        @pl.loop(0, _IDX_ROWS)
        def _(j):
            row = slot * _IDX_ROWS + j
            pltpu.async_copy(tbl_hbm.at[idx2.at[row]], rows2.at[row],
                             sems[slot])

    def wait_rows(slot):
        @pl.loop(0, _IDX_ROWS)
        def _(j):
            row = slot * _IDX_ROWS + j
            pltpu.make_async_copy(tbl_hbm.at[idx2.at[row]], rows2.at[row],
                                  sems[slot]).wait()

    def phase_b(l, slot):
        @pl.loop(0, _GROUPS)
        def _(g):
            rrow = slot * _IDX_ROWS + g
            rowv = col0 + rrow
            wbase = slot * _NROWS + g * 128
            acc0 = jnp.zeros((_LANES,), jnp.float32)
            acc1 = jnp.zeros((_LANES,), jnp.float32)
            for c in range(8):
                colv = iota + (c * _LANES)
                wordv = rem2[pl.ds(wbase + c * _LANES, _LANES)]
                f0 = plsc.load_gather(rows2, [rowv, colv, wordv])
                f1 = plsc.load_gather(rows2, [rowv, colv, wordv + 1])
                w = w2[pl.ds(wbase + c * _LANES, _LANES)]
                acc0 = acc0 + w * f0
                acc1 = acc1 + w * f1
            sidx = iota32 + (g * (_LANES * 32) + 2 * l)
            plsc.store_scatter(penc, [sidx], acc0)
            plsc.store_scatter(penc, [sidx + 1], acc1)

    @pl.loop(0, chunks)
    def _(ci):
        base = wid * ppw + ci * _P
        pltpu.sync_copy(texc_hbm.at[pl.ds(base * 3, _P * 3)], txyz)

        @pl.loop(0, _GROUPS)
        def _(g):
            gi = iota3 + g * (3 * _LANES)
            off = g * _LANES
            xb[pl.ds(off, _LANES)] = plsc.load_gather(txyz, [gi])
            yb[pl.ds(off, _LANES)] = plsc.load_gather(txyz, [gi + 1])
            zb[pl.ds(off, _LANES)] = plsc.load_gather(txyz, [gi + 2])

        phase_a(0, 0)
        fire(0)
        for l in range(1, _NUM_LEVELS):
            slot = l & 1
            phase_a(l, slot)
            fire(slot)
            wait_rows(1 - slot)
            phase_b(l - 1, 1 - slot)
        wait_rows(1)
        phase_b(_NUM_LEVELS - 1, 1)

        pltpu.sync_copy(penc, out_hbm.at[pl.ds(base * 32, _P * 32)])


@functools.lru_cache(maxsize=None)
def _build_encode(n_pts):
    assert n_pts % (_NW * _P) == 0
    mesh = plsc.VectorSubcoreMesh(core_axis_name="c", subcore_axis_name="s")
    return pl.kernel(
        _encode_body,
        out_type=jax.ShapeDtypeStruct((n_pts * 32,), jnp.float32),
        mesh=mesh,
        compiler_params=pltpu.CompilerParams(needs_layout_passes=False,
                                             use_tc_tiling_on_sc=False),
        scratch_types=[
            pltpu.VMEM((_P * 3,), jnp.float32),
            pltpu.VMEM((_P,), jnp.float32),
            pltpu.VMEM((_P,), jnp.float32),
            pltpu.VMEM((_P,), jnp.float32),
            pltpu.VMEM((2 * _IDX_ROWS, 128), jnp.int32),
            pltpu.VMEM((2 * _NROWS,), jnp.float32),
            pltpu.VMEM((2 * _NROWS,), jnp.int32),
            pltpu.VMEM((2 * _IDX_ROWS, 128, 8), jnp.float32),
            pltpu.VMEM((_P * 32,), jnp.float32),
            pltpu.SemaphoreType.DMA,
            pltpu.SemaphoreType.DMA,
        ],
    )


def _mlp_body(x_ref, w0_ref, w1_ref, w2_ref, o_ref):
    x = x_ref[...]
    h = jnp.maximum(jnp.dot(x, w0_ref[...], preferred_element_type=jnp.float32), 0.0)
    h = jnp.maximum(jnp.dot(h, w1_ref[...], preferred_element_type=jnp.float32), 0.0)
    o_ref[...] = jnp.dot(h, w2_ref[...], preferred_element_type=jnp.float32)


@functools.lru_cache(maxsize=None)
def _build_mlp(n_pts):
    blk = 8192
    assert n_pts % blk == 0
    return pl.pallas_call(
        _mlp_body,
        grid=(n_pts // blk,),
        in_specs=[
            pl.BlockSpec((blk, 32), lambda i: (i, 0)),
            pl.BlockSpec((32, 32), lambda i: (0, 0)),
            pl.BlockSpec((32, 32), lambda i: (0, 0)),
            pl.BlockSpec((32, 4), lambda i: (0, 0)),
        ],
        out_specs=pl.BlockSpec((blk, 4), lambda i: (i, 0)),
        out_shape=jax.ShapeDtypeStruct((n_pts, 4), jnp.float32),
    )


def kernel(texc, table, W0, W1, W2):
    x = texc.reshape(-1, 3).astype(jnp.float32)
    n_pts = x.shape[0]
    p0 = table[:, :, 0].reshape(-1)
    p1 = table[:, :, 1].reshape(-1)
    tbl_i = _build_repack()(p0, p1).reshape(_ENTRIES // 4, 8)
    p_enc = _build_encode(n_pts)(x.reshape(-1), tbl_i).reshape(n_pts, 32)
    return _build_mlp(n_pts)(p_enc, W0.T, W1.T, W2.T)
